# Initial kernel scaffold; baseline (speedup 1.0000x reference)
import jax
import jax.numpy as jnp
from jax.experimental import pallas as pl


def kernel(vertices, cat_id, params):
    raise NotImplementedError("write your pallas kernel here")


# speedup vs baseline: 3.5242x; 3.5242x over previous
import functools

import numpy as np
import jax
import jax.numpy as jnp
from jax.experimental import pallas as pl

_SUP = 7
_EPS = 1e-12
_F32 = jnp.float32

_NT = (((1,), (1,)), ((), ()))


def _ntdot(a, b):
    return jax.lax.dot_general(a, b, _NT, preferred_element_type=_F32)


def _nndot(a, b):
    return jnp.dot(a, b, preferred_element_type=_F32)


def _gdot(onehot, b):
    return jnp.dot(onehot, b, preferred_element_type=_F32,
                   precision=jax.lax.Precision.HIGHEST)


def _normalize_rows(x):
    n = jnp.sqrt(jnp.sum(x * x, axis=-1, keepdims=True))
    return x / jnp.maximum(n, _EPS)


def _pop_argmax(neg, iota):
    m = jnp.max(neg, axis=1, keepdims=True)
    eq = neg == m
    big = jnp.int32(neg.shape[1])
    amx = jnp.min(jnp.where(eq, iota, big), axis=1, keepdims=True)
    sel = iota == amx
    return sel.astype(neg.dtype), jnp.where(sel, -jnp.inf, neg)


def _neg_dist(feat_rows, feat_all, qrow, qcol):
    inner = _ntdot(feat_rows, feat_all)
    return -((-2.0 * inner + qrow) + qcol)


def _sum_supports(acc, out_ch):
    agg = acc[:, :out_ch]
    for s in range(1, _SUP):
        agg = agg + acc[:, s * out_ch:(s + 1) * out_ch]
    return agg


def _dir_normalize(dirs):
    n = jnp.sqrt(jnp.sum(dirs * dirs, axis=0, keepdims=True))
    return dirs / jnp.maximum(n, _EPS)


def _conv0_body(v_ref, qr_ref, qc_ref, steT_ref, dirs_ref, out_ref, *, n_nbr):
    v3 = v_ref[0]
    dn = _dir_normalize(dirs_ref[...])
    neg = _neg_dist(v3, v3, qr_ref[0], qc_ref[0])
    V = v3.shape[0]
    iota = jax.lax.broadcasted_iota(jnp.int32, (V, V), 1)
    _, neg = _pop_argmax(neg, iota)
    acc = jnp.full((V, dn.shape[1]), -jnp.inf, _F32)
    for _ in range(n_nbr):
        onehot, neg = _pop_argmax(neg, iota)
        nbr = _gdot(onehot, v3)
        rf = _normalize_rows(nbr - v3)
        theta = jax.nn.relu(_nndot(rf, dn))
        acc = jnp.maximum(acc, theta)
    agg = _sum_supports(acc, 128)
    f_ste = _nndot(v3, steT_ref[...])
    out_ref[0] = jax.nn.relu(agg + f_ste)


def _hs_layer_body(v_ref, fm_ref, qr_ref, qc_ref, steT_ref, dirs_ref, w_ref,
                   b_ref, out_ref, *, n_nbr, out_ch):
    v3 = v_ref[0]
    fm = fm_ref[0]
    dn = _dir_normalize(dirs_ref[...])
    neg = _neg_dist(fm, fm, qr_ref[0], qc_ref[0])
    V = fm.shape[0]
    iota = jax.lax.broadcasted_iota(jnp.int32, (V, V), 1)
    fmw = _nndot(fm, w_ref[...]) + b_ref[...]
    f_center = fmw[:, :out_ch]
    f_support = fmw[:, out_ch:]
    _, neg = _pop_argmax(neg, iota)
    acc = jnp.full((V, _SUP * out_ch), -jnp.inf, _F32)
    for _ in range(n_nbr):
        onehot, neg = _pop_argmax(neg, iota)
        nbr = _gdot(onehot, v3)
        rf = _normalize_rows(nbr - v3)
        theta = jax.nn.relu(_nndot(rf, dn))
        fs = _gdot(onehot, f_support)
        acc = jnp.maximum(acc, theta * fs)
    agg = _sum_supports(acc, out_ch)
    f_ste = _nndot(fm, steT_ref[...])
    out_ref[0] = f_center + agg + f_ste


def _bn_relu_body(x_ref, g_ref, b_ref, out_ref):
    x = x_ref[...]
    m = jnp.mean(x, axis=0, keepdims=True)
    var = jnp.mean((x - m) ** 2, axis=0, keepdims=True)
    out_ref[...] = jax.nn.relu(
        g_ref[...] * (x - m) / jnp.sqrt(var + 1e-5) + b_ref[...])


def _pool_body(vsel_ref, v_ref, fm_ref, qr_ref, qc_ref, out_ref, *, n_nbr):
    vsel = vsel_ref[0]
    v3 = v_ref[0]
    fm = fm_ref[0]
    neg = _neg_dist(vsel, v3, qr_ref[0], qc_ref[0])
    Vo, V = neg.shape
    iota = jax.lax.broadcasted_iota(jnp.int32, (Vo, V), 1)
    _, neg = _pop_argmax(neg, iota)
    acc = jnp.full((Vo, fm.shape[1]), -jnp.inf, _F32)
    for _ in range(n_nbr):
        onehot, neg = _pop_argmax(neg, iota)
        acc = jnp.maximum(acc, _gdot(onehot, fm))
    out_ref[0] = acc


def _near_gather_body(t_ref, s_ref, f_ref, qr_ref, qc_ref, out_ref):
    t = t_ref[0]
    s = s_ref[0]
    f = f_ref[0]
    inner = _ntdot(t, s)
    d = (qr_ref[0] + qc_ref[0]) - 2.0 * inner
    Vt, Vs = d.shape
    iota = jax.lax.broadcasted_iota(jnp.int32, (Vt, Vs), 1)
    mn = jnp.min(d, axis=1, keepdims=True)
    amn = jnp.min(jnp.where(d == mn, iota, jnp.int32(Vs)),
                  axis=1, keepdims=True)
    onehot = (iota == amn).astype(_F32)
    out_ref[0] = _gdot(onehot, f)


def _mlp_body(feat_ref, w1_ref, b1_ref, w2_ref, b2_ref, w3_ref, b3_ref,
              out_ref):
    h = jax.nn.relu(_nndot(feat_ref[0], w1_ref[...]) + b1_ref[...])
    h = jax.nn.relu(_nndot(h, w2_ref[...]) + b2_ref[...])
    h = jax.nn.relu(_nndot(h, w3_ref[...]) + b3_ref[...])
    out_ref[0] = h


def _batched_spec(shape):
    nd = len(shape)
    return pl.BlockSpec((1,) + shape[1:], lambda b: (b,) + (0,) * (nd - 1))


def _param_spec(shape):
    nd = len(shape)
    return pl.BlockSpec(shape, lambda b: (0,) * nd)


def _call_batched(body, batched_ins, param_ins, out_shape):
    B = batched_ins[0].shape[0]
    in_specs = ([_batched_spec(x.shape) for x in batched_ins]
                + [_param_spec(p.shape) for p in param_ins])
    return pl.pallas_call(
        body,
        grid=(B,),
        in_specs=in_specs,
        out_specs=_batched_spec(out_shape),
        out_shape=jax.ShapeDtypeStruct(out_shape, _F32),
    )(*batched_ins, *param_ins)


def _quads(x):
    q = jnp.sum(x * x, axis=2)
    return q[:, None, :], q[:, :, None]


def _bn_relu(x, g, b):
    B, V, C = x.shape
    x2 = x.reshape(B * V, C)
    out = pl.pallas_call(
        _bn_relu_body,
        out_shape=jax.ShapeDtypeStruct((B * V, C), _F32),
    )(x2, g.reshape(1, C), b.reshape(1, C))
    return out.reshape(B, V, C)


def _hs_layer(v, fm, p, n_nbr, out_ch):
    B, V, _ = v.shape
    qr, qc = _quads(fm)
    body = functools.partial(_hs_layer_body, n_nbr=n_nbr, out_ch=out_ch)
    return _call_batched(
        body, [v, fm, qr, qc],
        [p['ste'].T, p['directions'], p['weights'],
         p['bias'].reshape(1, -1)],
        (B, V, out_ch))


def _pool(vsel, v, fm, n_nbr):
    B = v.shape[0]
    qs = jnp.sum(vsel * vsel, axis=2)
    qv = jnp.sum(v * v, axis=2)
    return _call_batched(
        functools.partial(_pool_body, n_nbr=n_nbr),
        [vsel, v, fm, qv[:, None, :], qs[:, :, None]], [],
        (B, vsel.shape[1], fm.shape[2]))


def _near_gather(t, s, f):
    B = t.shape[0]
    s2 = jnp.sum(s * s, axis=2)
    t2 = jnp.sum(t * t, axis=2)
    return _call_batched(
        _near_gather_body,
        [t, s, f, s2[:, None, :], t2[:, :, None]], [],
        (B, t.shape[1], f.shape[2]))


def kernel(vertices, cat_id, params):
    B, V, _ = vertices.shape
    p = params
    vertices = vertices.astype(_F32)

    qr0, qc0 = _quads(vertices)
    fm0 = _call_batched(
        functools.partial(_conv0_body, n_nbr=10),
        [vertices, qr0, qc0],
        [p['conv0']['ste'].T, p['conv0']['directions']],
        (B, V, 128))

    c1 = _hs_layer(vertices, fm0, p['conv1'], 10, 128)
    fm1 = _bn_relu(c1, p['bn1_g'], p['bn1_b'])

    idx1 = np.random.RandomState(0).permutation(V)[: V // 4]
    v1 = vertices[:, idx1, :]
    f1 = _pool(v1, vertices, fm1, 4)

    n23 = min(10, (V // 4) // 8)

    c2 = _hs_layer(v1, f1, p['conv2'], n23, 256)
    fm2 = _bn_relu(c2, p['bn2_g'], p['bn2_b'])

    c3 = _hs_layer(v1, fm2, p['conv3'], n23, 256)
    fm3 = _bn_relu(c3, p['bn3_g'], p['bn3_b'])

    idx2 = np.random.RandomState(1).permutation(V // 4)[: V // 16]
    v2 = v1[:, idx2, :]
    f2 = _pool(v2, v1, fm3, 4)

    n4 = min(10, (V // 16) // 8)

    fm4 = _hs_layer(v2, f2, p['conv4'], n4, 512)

    g23 = _near_gather(vertices, v1, jnp.concatenate([fm2, fm3], axis=2))
    g4 = _near_gather(vertices, v2, fm4)

    oh = jax.nn.one_hot(cat_id.reshape(-1), 6, dtype=_F32)
    ohb = jnp.broadcast_to(oh[:, None, :], (B, V, 6))
    feat = jnp.concatenate([fm0, fm1, g23, g4, ohb], axis=2)

    h = _call_batched(
        _mlp_body, [feat],
        [p['mlp_w1'].T, p['mlp_b1'].reshape(1, -1),
         p['mlp_w2'].T, p['mlp_b2'].reshape(1, -1),
         p['mlp_w3'].T, p['mlp_b3'].reshape(1, -1)],
        (B, V, 128))
    return jnp.transpose(h, (0, 2, 1))


# speedup vs baseline: 4.2644x; 1.2101x over previous
import functools

import numpy as np
import jax
import jax.numpy as jnp
from jax.experimental import pallas as pl

_SUP = 7
_EPS = 1e-12
_F32 = jnp.float32

_NT = (((1,), (1,)), ((), ()))


def _ntdot(a, b):
    return jax.lax.dot_general(a, b, _NT, preferred_element_type=_F32)


def _nndot(a, b):
    return jnp.dot(a, b, preferred_element_type=_F32)


def _gdot(onehot, b):
    return jnp.dot(onehot, b, preferred_element_type=_F32,
                   precision=jax.lax.Precision.HIGHEST)


def _normalize_rows(x):
    n = jnp.sqrt(jnp.sum(x * x, axis=-1, keepdims=True))
    return x / jnp.maximum(n, _EPS)


def _pop_argmax(neg, iota):
    m = jnp.max(neg, axis=1, keepdims=True)
    eq = neg == m
    big = jnp.int32(neg.shape[1])
    amx = jnp.min(jnp.where(eq, iota, big), axis=1, keepdims=True)
    sel = iota == amx
    return sel.astype(neg.dtype), jnp.where(sel, -jnp.inf, neg)


def _neg_dist(feat_rows, feat_all, qrow, qcol):
    inner = _ntdot(feat_rows, feat_all)
    return -((-2.0 * inner + qrow) + qcol)


def _sum_supports(acc, out_ch):
    agg = acc[:, :out_ch]
    for s in range(1, _SUP):
        agg = agg + acc[:, s * out_ch:(s + 1) * out_ch]
    return agg


def _dir_normalize(dirs):
    n = jnp.sqrt(jnp.sum(dirs * dirs, axis=0, keepdims=True))
    return dirs / jnp.maximum(n, _EPS)


def _conv0_body(v_ref, qr_ref, qc_ref, steT_ref, dirs_ref, out_ref, *, n_nbr):
    v3 = v_ref[0]
    dn = _dir_normalize(dirs_ref[...])
    neg = _neg_dist(v3, v3, qr_ref[0], qc_ref[0])
    V = v3.shape[0]
    iota = jax.lax.broadcasted_iota(jnp.int32, (V, V), 1)
    _, neg = _pop_argmax(neg, iota)
    acc = jnp.full((V, dn.shape[1]), -jnp.inf, _F32)
    for _ in range(n_nbr):
        onehot, neg = _pop_argmax(neg, iota)
        nbr = _gdot(onehot, v3)
        rf = _normalize_rows(nbr - v3)
        theta = jax.nn.relu(_nndot(rf, dn))
        acc = jnp.maximum(acc, theta)
    agg = _sum_supports(acc, 128)
    f_ste = _nndot(v3, steT_ref[...])
    out_ref[0] = jax.nn.relu(agg + f_ste)


def _hs_layer_body(v_ref, fm_ref, qr_ref, qc_ref, steT_ref, dirs_ref, w_ref,
                   b_ref, out_ref, *, n_nbr, out_ch):
    v3 = v_ref[0]
    fm = fm_ref[0]
    dn = _dir_normalize(dirs_ref[...])
    neg = _neg_dist(fm, fm, qr_ref[0], qc_ref[0])
    V = fm.shape[0]
    iota = jax.lax.broadcasted_iota(jnp.int32, (V, V), 1)
    w = w_ref[...]
    b = b_ref[...]
    f_center = _nndot(fm, w[:, :out_ch]) + b[:, :out_ch]
    w_sup = w[:, out_ch:]
    b_sup = b[:, out_ch:]
    _, neg = _pop_argmax(neg, iota)
    acc = jnp.full((V, _SUP * out_ch), -jnp.inf, _F32)
    for _ in range(n_nbr):
        onehot, neg = _pop_argmax(neg, iota)
        nbr = _gdot(onehot, v3)
        rf = _normalize_rows(nbr - v3)
        theta = jax.nn.relu(_nndot(rf, dn))
        fs = _nndot(_gdot(onehot, fm), w_sup) + b_sup
        acc = jnp.maximum(acc, theta * fs)
    agg = _sum_supports(acc, out_ch)
    f_ste = _nndot(fm, steT_ref[...])
    out_ref[0] = f_center + agg + f_ste


def _bn_relu_body(x_ref, g_ref, b_ref, out_ref):
    x = x_ref[...]
    m = jnp.mean(x, axis=0, keepdims=True)
    var = jnp.mean((x - m) ** 2, axis=0, keepdims=True)
    out_ref[...] = jax.nn.relu(
        g_ref[...] * (x - m) / jnp.sqrt(var + 1e-5) + b_ref[...])


def _pool_body(vsel_ref, v_ref, fm_ref, qr_ref, qc_ref, out_ref, *, n_nbr):
    vsel = vsel_ref[0]
    v3 = v_ref[0]
    fm = fm_ref[0]
    neg = _neg_dist(vsel, v3, qr_ref[0], qc_ref[0])
    Vo, V = neg.shape
    iota = jax.lax.broadcasted_iota(jnp.int32, (Vo, V), 1)
    _, neg = _pop_argmax(neg, iota)
    acc = jnp.full((Vo, fm.shape[1]), -jnp.inf, _F32)
    for _ in range(n_nbr):
        onehot, neg = _pop_argmax(neg, iota)
        acc = jnp.maximum(acc, _gdot(onehot, fm))
    out_ref[0] = acc


def _near_gather_body(t_ref, s_ref, f_ref, qr_ref, qc_ref, out_ref):
    t = t_ref[0]
    s = s_ref[0]
    f = f_ref[0]
    inner = _ntdot(t, s)
    d = (qr_ref[0] + qc_ref[0]) - 2.0 * inner
    Vt, Vs = d.shape
    iota = jax.lax.broadcasted_iota(jnp.int32, (Vt, Vs), 1)
    mn = jnp.min(d, axis=1, keepdims=True)
    amn = jnp.min(jnp.where(d == mn, iota, jnp.int32(Vs)),
                  axis=1, keepdims=True)
    onehot = (iota == amn).astype(_F32)
    out_ref[0] = _gdot(onehot, f)


def _mlp_body(feat_ref, w1_ref, b1_ref, w2_ref, b2_ref, w3_ref, b3_ref,
              out_ref):
    h = jax.nn.relu(_nndot(feat_ref[0], w1_ref[...]) + b1_ref[...])
    h = jax.nn.relu(_nndot(h, w2_ref[...]) + b2_ref[...])
    h = jax.nn.relu(_nndot(h, w3_ref[...]) + b3_ref[...])
    out_ref[0] = h


def _batched_spec(shape):
    nd = len(shape)
    return pl.BlockSpec((1,) + shape[1:], lambda b: (b,) + (0,) * (nd - 1))


def _param_spec(shape):
    nd = len(shape)
    return pl.BlockSpec(shape, lambda b: (0,) * nd)


def _call_batched(body, batched_ins, param_ins, out_shape):
    B = batched_ins[0].shape[0]
    in_specs = ([_batched_spec(x.shape) for x in batched_ins]
                + [_param_spec(p.shape) for p in param_ins])
    return pl.pallas_call(
        body,
        grid=(B,),
        in_specs=in_specs,
        out_specs=_batched_spec(out_shape),
        out_shape=jax.ShapeDtypeStruct(out_shape, _F32),
    )(*batched_ins, *param_ins)


def _quads(x):
    q = jnp.sum(x * x, axis=2)
    return q[:, None, :], q[:, :, None]


def _bn_relu(x, g, b):
    B, V, C = x.shape
    x2 = x.reshape(B * V, C)
    out = pl.pallas_call(
        _bn_relu_body,
        out_shape=jax.ShapeDtypeStruct((B * V, C), _F32),
    )(x2, g.reshape(1, C), b.reshape(1, C))
    return out.reshape(B, V, C)


def _hs_layer(v, fm, p, n_nbr, out_ch):
    B, V, _ = v.shape
    qr, qc = _quads(fm)
    body = functools.partial(_hs_layer_body, n_nbr=n_nbr, out_ch=out_ch)
    return _call_batched(
        body, [v, fm, qr, qc],
        [p['ste'].T, p['directions'], p['weights'],
         p['bias'].reshape(1, -1)],
        (B, V, out_ch))


def _pool(vsel, v, fm, n_nbr):
    B = v.shape[0]
    qs = jnp.sum(vsel * vsel, axis=2)
    qv = jnp.sum(v * v, axis=2)
    return _call_batched(
        functools.partial(_pool_body, n_nbr=n_nbr),
        [vsel, v, fm, qv[:, None, :], qs[:, :, None]], [],
        (B, vsel.shape[1], fm.shape[2]))


def _near_gather(t, s, f):
    B = t.shape[0]
    s2 = jnp.sum(s * s, axis=2)
    t2 = jnp.sum(t * t, axis=2)
    return _call_batched(
        _near_gather_body,
        [t, s, f, s2[:, None, :], t2[:, :, None]], [],
        (B, t.shape[1], f.shape[2]))


def kernel(vertices, cat_id, params):
    B, V, _ = vertices.shape
    p = params
    vertices = vertices.astype(_F32)

    qr0, qc0 = _quads(vertices)
    fm0 = _call_batched(
        functools.partial(_conv0_body, n_nbr=10),
        [vertices, qr0, qc0],
        [p['conv0']['ste'].T, p['conv0']['directions']],
        (B, V, 128))

    c1 = _hs_layer(vertices, fm0, p['conv1'], 10, 128)
    fm1 = _bn_relu(c1, p['bn1_g'], p['bn1_b'])

    idx1 = np.random.RandomState(0).permutation(V)[: V // 4]
    v1 = vertices[:, idx1, :]
    f1 = _pool(v1, vertices, fm1, 4)

    n23 = min(10, (V // 4) // 8)

    c2 = _hs_layer(v1, f1, p['conv2'], n23, 256)
    fm2 = _bn_relu(c2, p['bn2_g'], p['bn2_b'])

    c3 = _hs_layer(v1, fm2, p['conv3'], n23, 256)
    fm3 = _bn_relu(c3, p['bn3_g'], p['bn3_b'])

    idx2 = np.random.RandomState(1).permutation(V // 4)[: V // 16]
    v2 = v1[:, idx2, :]
    f2 = _pool(v2, v1, fm3, 4)

    n4 = min(10, (V // 16) // 8)

    fm4 = _hs_layer(v2, f2, p['conv4'], n4, 512)

    g23 = _near_gather(vertices, v1, jnp.concatenate([fm2, fm3], axis=2))
    g4 = _near_gather(vertices, v2, fm4)

    oh = jax.nn.one_hot(cat_id.reshape(-1), 6, dtype=_F32)
    ohb = jnp.broadcast_to(oh[:, None, :], (B, V, 6))
    feat = jnp.concatenate([fm0, fm1, g23, g4, ohb], axis=2)

    h = _call_batched(
        _mlp_body, [feat],
        [p['mlp_w1'].T, p['mlp_b1'].reshape(1, -1),
         p['mlp_w2'].T, p['mlp_b2'].reshape(1, -1),
         p['mlp_w3'].T, p['mlp_b3'].reshape(1, -1)],
        (B, V, 128))
    return jnp.transpose(h, (0, 2, 1))


# speedup vs baseline: 6.8562x; 1.6078x over previous
import functools

import numpy as np
import jax
import jax.numpy as jnp
from jax.experimental import pallas as pl

_SUP = 7
_EPS = 1e-12
_F32 = jnp.float32

_NT = (((1,), (1,)), ((), ()))


def _ntdot(a, b):
    return jax.lax.dot_general(a, b, _NT, preferred_element_type=_F32)


def _nndot(a, b):
    return jnp.dot(a, b, preferred_element_type=_F32)


def _gdot(onehot, b):
    oh = onehot.astype(jnp.bfloat16)
    b1 = b.astype(jnp.bfloat16)
    r1 = b - b1.astype(_F32)
    b2 = r1.astype(jnp.bfloat16)
    b3 = (r1 - b2.astype(_F32)).astype(jnp.bfloat16)
    g1 = jnp.dot(oh, b1, preferred_element_type=_F32)
    g2 = jnp.dot(oh, b2, preferred_element_type=_F32)
    g3 = jnp.dot(oh, b3, preferred_element_type=_F32)
    return (g1 + g2) + g3


def _normalize_rows(x):
    n = jnp.sqrt(jnp.sum(x * x, axis=-1, keepdims=True))
    return x / jnp.maximum(n, _EPS)


def _pop_argmax(neg, iota):
    m = jnp.max(neg, axis=1, keepdims=True)
    eq = neg == m
    big = jnp.int32(neg.shape[1])
    amx = jnp.min(jnp.where(eq, iota, big), axis=1, keepdims=True)
    sel = iota == amx
    return sel.astype(neg.dtype), jnp.where(sel, -jnp.inf, neg)


def _neg_dist(feat_rows, feat_all, qrow, qcol):
    inner = _ntdot(feat_rows, feat_all)
    return -((-2.0 * inner + qrow) + qcol)


def _sum_supports(acc, out_ch):
    agg = acc[:, :out_ch]
    for s in range(1, _SUP):
        agg = agg + acc[:, s * out_ch:(s + 1) * out_ch]
    return agg


def _dir_normalize(dirs):
    n = jnp.sqrt(jnp.sum(dirs * dirs, axis=0, keepdims=True))
    return dirs / jnp.maximum(n, _EPS)


def _conv0_body(v_ref, qr_ref, qc_ref, steT_ref, dirs_ref, out_ref, *, n_nbr):
    v3 = v_ref[0]
    dn = _dir_normalize(dirs_ref[...])
    neg = _neg_dist(v3, v3, qr_ref[0], qc_ref[0])
    V = v3.shape[0]
    iota = jax.lax.broadcasted_iota(jnp.int32, (V, V), 1)
    _, neg = _pop_argmax(neg, iota)
    acc = jnp.full((V, dn.shape[1]), -jnp.inf, _F32)
    for _ in range(n_nbr):
        onehot, neg = _pop_argmax(neg, iota)
        nbr = _gdot(onehot, v3)
        rf = _normalize_rows(nbr - v3)
        theta = jax.nn.relu(_nndot(rf, dn))
        acc = jnp.maximum(acc, theta)
    agg = _sum_supports(acc, 128)
    f_ste = _nndot(v3, steT_ref[...])
    out_ref[0] = jax.nn.relu(agg + f_ste)


def _hs_layer_body(v_ref, fm_ref, qr_ref, qc_ref, steT_ref, dirs_ref, w_ref,
                   b_ref, out_ref, *, n_nbr, out_ch):
    v3 = v_ref[0]
    fm = fm_ref[0]
    dn = _dir_normalize(dirs_ref[...])
    neg = _neg_dist(fm, fm, qr_ref[0], qc_ref[0])
    V = fm.shape[0]
    iota = jax.lax.broadcasted_iota(jnp.int32, (V, V), 1)
    w = w_ref[...]
    b = b_ref[...]
    f_center = _nndot(fm, w[:, :out_ch]) + b[:, :out_ch]
    w_sup = w[:, out_ch:]
    b_sup = b[:, out_ch:]
    _, neg = _pop_argmax(neg, iota)
    acc = jnp.full((V, _SUP * out_ch), -jnp.inf, _F32)
    for _ in range(n_nbr):
        onehot, neg = _pop_argmax(neg, iota)
        nbr = _gdot(onehot, v3)
        rf = _normalize_rows(nbr - v3)
        theta = jax.nn.relu(_nndot(rf, dn))
        fs = _nndot(_gdot(onehot, fm), w_sup) + b_sup
        acc = jnp.maximum(acc, theta * fs)
    agg = _sum_supports(acc, out_ch)
    f_ste = _nndot(fm, steT_ref[...])
    out_ref[0] = f_center + agg + f_ste


def _bn_relu_body(x_ref, g_ref, b_ref, out_ref):
    x = x_ref[...]
    m = jnp.mean(x, axis=0, keepdims=True)
    var = jnp.mean((x - m) ** 2, axis=0, keepdims=True)
    out_ref[...] = jax.nn.relu(
        g_ref[...] * (x - m) / jnp.sqrt(var + 1e-5) + b_ref[...])


def _pool_body(vsel_ref, v_ref, fm_ref, qr_ref, qc_ref, out_ref, *, n_nbr):
    vsel = vsel_ref[0]
    v3 = v_ref[0]
    fm = fm_ref[0]
    neg = _neg_dist(vsel, v3, qr_ref[0], qc_ref[0])
    Vo, V = neg.shape
    iota = jax.lax.broadcasted_iota(jnp.int32, (Vo, V), 1)
    _, neg = _pop_argmax(neg, iota)
    acc = jnp.full((Vo, fm.shape[1]), -jnp.inf, _F32)
    for _ in range(n_nbr):
        onehot, neg = _pop_argmax(neg, iota)
        acc = jnp.maximum(acc, _gdot(onehot, fm))
    out_ref[0] = acc


def _near_gather_body(t_ref, s_ref, f_ref, qr_ref, qc_ref, out_ref):
    t = t_ref[0]
    s = s_ref[0]
    f = f_ref[0]
    inner = _ntdot(t, s)
    d = (qr_ref[0] + qc_ref[0]) - 2.0 * inner
    Vt, Vs = d.shape
    iota = jax.lax.broadcasted_iota(jnp.int32, (Vt, Vs), 1)
    mn = jnp.min(d, axis=1, keepdims=True)
    amn = jnp.min(jnp.where(d == mn, iota, jnp.int32(Vs)),
                  axis=1, keepdims=True)
    onehot = (iota == amn).astype(_F32)
    out_ref[0] = _gdot(onehot, f)


def _mlp_body(feat_ref, w1_ref, b1_ref, w2_ref, b2_ref, w3_ref, b3_ref,
              out_ref):
    h = jax.nn.relu(_nndot(feat_ref[0], w1_ref[...]) + b1_ref[...])
    h = jax.nn.relu(_nndot(h, w2_ref[...]) + b2_ref[...])
    h = jax.nn.relu(_nndot(h, w3_ref[...]) + b3_ref[...])
    out_ref[0] = h


def _batched_spec(shape):
    nd = len(shape)
    return pl.BlockSpec((1,) + shape[1:], lambda b: (b,) + (0,) * (nd - 1))


def _param_spec(shape):
    nd = len(shape)
    return pl.BlockSpec(shape, lambda b: (0,) * nd)


def _call_batched(body, batched_ins, param_ins, out_shape):
    B = batched_ins[0].shape[0]
    in_specs = ([_batched_spec(x.shape) for x in batched_ins]
                + [_param_spec(p.shape) for p in param_ins])
    return pl.pallas_call(
        body,
        grid=(B,),
        in_specs=in_specs,
        out_specs=_batched_spec(out_shape),
        out_shape=jax.ShapeDtypeStruct(out_shape, _F32),
    )(*batched_ins, *param_ins)


def _quads(x):
    q = jnp.sum(x * x, axis=2)
    return q[:, None, :], q[:, :, None]


def _bn_relu(x, g, b):
    B, V, C = x.shape
    x2 = x.reshape(B * V, C)
    out = pl.pallas_call(
        _bn_relu_body,
        out_shape=jax.ShapeDtypeStruct((B * V, C), _F32),
    )(x2, g.reshape(1, C), b.reshape(1, C))
    return out.reshape(B, V, C)


def _hs_layer(v, fm, p, n_nbr, out_ch):
    B, V, _ = v.shape
    qr, qc = _quads(fm)
    body = functools.partial(_hs_layer_body, n_nbr=n_nbr, out_ch=out_ch)
    return _call_batched(
        body, [v, fm, qr, qc],
        [p['ste'].T, p['directions'], p['weights'],
         p['bias'].reshape(1, -1)],
        (B, V, out_ch))


def _pool(vsel, v, fm, n_nbr):
    B = v.shape[0]
    qs = jnp.sum(vsel * vsel, axis=2)
    qv = jnp.sum(v * v, axis=2)
    return _call_batched(
        functools.partial(_pool_body, n_nbr=n_nbr),
        [vsel, v, fm, qv[:, None, :], qs[:, :, None]], [],
        (B, vsel.shape[1], fm.shape[2]))


def _near_gather(t, s, f):
    B = t.shape[0]
    s2 = jnp.sum(s * s, axis=2)
    t2 = jnp.sum(t * t, axis=2)
    return _call_batched(
        _near_gather_body,
        [t, s, f, s2[:, None, :], t2[:, :, None]], [],
        (B, t.shape[1], f.shape[2]))


def kernel(vertices, cat_id, params):
    B, V, _ = vertices.shape
    p = params
    vertices = vertices.astype(_F32)

    qr0, qc0 = _quads(vertices)
    fm0 = _call_batched(
        functools.partial(_conv0_body, n_nbr=10),
        [vertices, qr0, qc0],
        [p['conv0']['ste'].T, p['conv0']['directions']],
        (B, V, 128))

    c1 = _hs_layer(vertices, fm0, p['conv1'], 10, 128)
    fm1 = _bn_relu(c1, p['bn1_g'], p['bn1_b'])

    idx1 = np.random.RandomState(0).permutation(V)[: V // 4]
    v1 = vertices[:, idx1, :]
    f1 = _pool(v1, vertices, fm1, 4)

    n23 = min(10, (V // 4) // 8)

    c2 = _hs_layer(v1, f1, p['conv2'], n23, 256)
    fm2 = _bn_relu(c2, p['bn2_g'], p['bn2_b'])

    c3 = _hs_layer(v1, fm2, p['conv3'], n23, 256)
    fm3 = _bn_relu(c3, p['bn3_g'], p['bn3_b'])

    idx2 = np.random.RandomState(1).permutation(V // 4)[: V // 16]
    v2 = v1[:, idx2, :]
    f2 = _pool(v2, v1, fm3, 4)

    n4 = min(10, (V // 16) // 8)

    fm4 = _hs_layer(v2, f2, p['conv4'], n4, 512)

    g23 = _near_gather(vertices, v1, jnp.concatenate([fm2, fm3], axis=2))
    g4 = _near_gather(vertices, v2, fm4)

    oh = jax.nn.one_hot(cat_id.reshape(-1), 6, dtype=_F32)
    ohb = jnp.broadcast_to(oh[:, None, :], (B, V, 6))
    feat = jnp.concatenate([fm0, fm1, g23, g4, ohb], axis=2)

    h = _call_batched(
        _mlp_body, [feat],
        [p['mlp_w1'].T, p['mlp_b1'].reshape(1, -1),
         p['mlp_w2'].T, p['mlp_b2'].reshape(1, -1),
         p['mlp_w3'].T, p['mlp_b3'].reshape(1, -1)],
        (B, V, 128))
    return jnp.transpose(h, (0, 2, 1))


# speedup vs baseline: 7.0679x; 1.0309x over previous
import functools

import numpy as np
import jax
import jax.numpy as jnp
from jax.experimental import pallas as pl

_SUP = 7
_EPS = 1e-12
_F32 = jnp.float32

_NT = (((1,), (1,)), ((), ()))


def _ntdot(a, b):
    return jax.lax.dot_general(a, b, _NT, preferred_element_type=_F32)


def _nndot(a, b):
    return jnp.dot(a, b, preferred_element_type=_F32)


def _gdot(onehot, b):
    oh = onehot.astype(jnp.bfloat16)
    b1 = b.astype(jnp.bfloat16)
    r1 = b - b1.astype(_F32)
    b2 = r1.astype(jnp.bfloat16)
    b3 = (r1 - b2.astype(_F32)).astype(jnp.bfloat16)
    g1 = jnp.dot(oh, b1, preferred_element_type=_F32)
    g2 = jnp.dot(oh, b2, preferred_element_type=_F32)
    g3 = jnp.dot(oh, b3, preferred_element_type=_F32)
    return (g1 + g2) + g3


def _normalize_rows(x):
    n = jnp.sqrt(jnp.sum(x * x, axis=-1, keepdims=True))
    return x / jnp.maximum(n, _EPS)


def _pop_argmax(neg, iota):
    m = jnp.max(neg, axis=1, keepdims=True)
    eq = neg == m
    big = jnp.int32(neg.shape[1])
    amx = jnp.min(jnp.where(eq, iota, big), axis=1, keepdims=True)
    sel = iota == amx
    return sel.astype(neg.dtype), jnp.where(sel, -jnp.inf, neg)


def _neg_dist(feat_rows, feat_all, qrow, qcol):
    inner = _ntdot(feat_rows, feat_all)
    return -((-2.0 * inner + qrow) + qcol)


def _sum_supports(acc, out_ch):
    agg = acc[:, :out_ch]
    for s in range(1, _SUP):
        agg = agg + acc[:, s * out_ch:(s + 1) * out_ch]
    return agg


def _dir_normalize(dirs):
    n = jnp.sqrt(jnp.sum(dirs * dirs, axis=0, keepdims=True))
    return dirs / jnp.maximum(n, _EPS)


def _conv0_body(v_ref, qr_ref, qc_ref, steT_ref, dirs_ref, out_ref, *, n_nbr):
    v3 = v_ref[0]
    dn = _dir_normalize(dirs_ref[...])
    neg = _neg_dist(v3, v3, qr_ref[0], qc_ref[0])
    V = v3.shape[0]
    iota = jax.lax.broadcasted_iota(jnp.int32, (V, V), 1)
    _, neg = _pop_argmax(neg, iota)
    acc = jnp.full((V, dn.shape[1]), -jnp.inf, _F32)
    for _ in range(n_nbr):
        onehot, neg = _pop_argmax(neg, iota)
        nbr = _gdot(onehot, v3)
        rf = _normalize_rows(nbr - v3)
        theta = jax.nn.relu(_nndot(rf, dn))
        acc = jnp.maximum(acc, theta)
    agg = _sum_supports(acc, 128)
    f_ste = _nndot(v3, steT_ref[...])
    out_ref[0] = jax.nn.relu(agg + f_ste)


def _hs_layer_body(v_ref, fm_ref, qr_ref, qc_ref, steT_ref, dirs_ref, w_ref,
                   b_ref, out_ref, *, n_nbr, out_ch):
    v3 = v_ref[0]
    fm = fm_ref[0]
    dn = _dir_normalize(dirs_ref[...])
    neg = _neg_dist(fm, fm, qr_ref[0], qc_ref[0])
    V = fm.shape[0]
    iota = jax.lax.broadcasted_iota(jnp.int32, (V, V), 1)
    w = w_ref[...]
    b = b_ref[...]
    f_center = _nndot(fm, w[:, :out_ch]) + b[:, :out_ch]
    w_sup = w[:, out_ch:]
    b_sup = b[:, out_ch:]
    _, neg = _pop_argmax(neg, iota)
    acc = jnp.full((V, _SUP * out_ch), -jnp.inf, _F32)
    for _ in range(n_nbr):
        onehot, neg = _pop_argmax(neg, iota)
        nbr = _gdot(onehot, v3)
        rf = _normalize_rows(nbr - v3)
        theta = jax.nn.relu(_nndot(rf, dn))
        fs = _nndot(_gdot(onehot, fm), w_sup) + b_sup
        acc = jnp.maximum(acc, theta * fs)
    agg = _sum_supports(acc, out_ch)
    f_ste = _nndot(fm, steT_ref[...])
    out_ref[0] = f_center + agg + f_ste


def _bn_relu_body(x_ref, g_ref, b_ref, out_ref):
    x = x_ref[...]
    m = jnp.mean(x, axis=0, keepdims=True)
    var = jnp.mean((x - m) ** 2, axis=0, keepdims=True)
    out_ref[...] = jax.nn.relu(
        g_ref[...] * (x - m) / jnp.sqrt(var + 1e-5) + b_ref[...])


def _pool_body(vsel_ref, v_ref, fm_ref, qr_ref, qc_ref, out_ref, *, n_nbr):
    vsel = vsel_ref[0]
    v3 = v_ref[0]
    fm = fm_ref[0]
    neg = _neg_dist(vsel, v3, qr_ref[0], qc_ref[0])
    Vo, V = neg.shape
    iota = jax.lax.broadcasted_iota(jnp.int32, (Vo, V), 1)
    _, neg = _pop_argmax(neg, iota)
    acc = jnp.full((Vo, fm.shape[1]), -jnp.inf, _F32)
    for _ in range(n_nbr):
        onehot, neg = _pop_argmax(neg, iota)
        acc = jnp.maximum(acc, _gdot(onehot, fm))
    out_ref[0] = acc


def _nearest_onehot(t, s, qr, qc):
    inner = _ntdot(t, s)
    d = (qr + qc) - 2.0 * inner
    Vt, Vs = d.shape
    iota = jax.lax.broadcasted_iota(jnp.int32, (Vt, Vs), 1)
    mn = jnp.min(d, axis=1, keepdims=True)
    amn = jnp.min(jnp.where(d == mn, iota, jnp.int32(Vs)),
                  axis=1, keepdims=True)
    return (iota == amn).astype(_F32)


def _head_body(t_ref, s1_ref, s2_ref, q1r_ref, q2r_ref, qtc_ref,
               fm0_ref, fm1_ref, fm2_ref, fm3_ref, fm4_ref, oh_ref,
               w1a_ref, w1b_ref, w1c_ref, w1d_ref, w1e_ref, w1f_ref, b1_ref,
               w2_ref, b2_ref, w3_ref, b3_ref, out_ref):
    t = t_ref[0]
    qtc = qtc_ref[0]
    oh1 = _nearest_onehot(t, s1_ref[0], q1r_ref[0], qtc)
    oh2 = _nearest_onehot(t, s2_ref[0], q2r_ref[0], qtc)
    g2 = _gdot(oh1, fm2_ref[0])
    g3 = _gdot(oh1, fm3_ref[0])
    g4 = _gdot(oh2, fm4_ref[0])
    h = (_nndot(fm0_ref[0], w1a_ref[...]) + _nndot(fm1_ref[0], w1b_ref[...])
         + _nndot(g2, w1c_ref[...]) + _nndot(g3, w1d_ref[...])
         + _nndot(g4, w1e_ref[...]) + _nndot(oh_ref[0], w1f_ref[...])
         + b1_ref[...])
    h = jax.nn.relu(h)
    h = jax.nn.relu(_nndot(h, w2_ref[...]) + b2_ref[...])
    h = jax.nn.relu(_nndot(h, w3_ref[...]) + b3_ref[...])
    out_ref[0] = h


def _batched_spec(shape):
    nd = len(shape)
    return pl.BlockSpec((1,) + shape[1:], lambda b: (b,) + (0,) * (nd - 1))


def _param_spec(shape):
    nd = len(shape)
    return pl.BlockSpec(shape, lambda b: (0,) * nd)


def _call_batched(body, batched_ins, param_ins, out_shape):
    B = batched_ins[0].shape[0]
    in_specs = ([_batched_spec(x.shape) for x in batched_ins]
                + [_param_spec(p.shape) for p in param_ins])
    return pl.pallas_call(
        body,
        grid=(B,),
        in_specs=in_specs,
        out_specs=_batched_spec(out_shape),
        out_shape=jax.ShapeDtypeStruct(out_shape, _F32),
    )(*batched_ins, *param_ins)


def _quads(x):
    q = jnp.sum(x * x, axis=2)
    return q[:, None, :], q[:, :, None]


def _bn_relu(x, g, b):
    B, V, C = x.shape
    x2 = x.reshape(B * V, C)
    out = pl.pallas_call(
        _bn_relu_body,
        out_shape=jax.ShapeDtypeStruct((B * V, C), _F32),
    )(x2, g.reshape(1, C), b.reshape(1, C))
    return out.reshape(B, V, C)


def _hs_layer(v, fm, p, n_nbr, out_ch):
    B, V, _ = v.shape
    qr, qc = _quads(fm)
    body = functools.partial(_hs_layer_body, n_nbr=n_nbr, out_ch=out_ch)
    return _call_batched(
        body, [v, fm, qr, qc],
        [p['ste'].T, p['directions'], p['weights'],
         p['bias'].reshape(1, -1)],
        (B, V, out_ch))


def _pool(vsel, v, fm, n_nbr):
    B = v.shape[0]
    qs = jnp.sum(vsel * vsel, axis=2)
    qv = jnp.sum(v * v, axis=2)
    return _call_batched(
        functools.partial(_pool_body, n_nbr=n_nbr),
        [vsel, v, fm, qv[:, None, :], qs[:, :, None]], [],
        (B, vsel.shape[1], fm.shape[2]))


def kernel(vertices, cat_id, params):
    B, V, _ = vertices.shape
    p = params
    vertices = vertices.astype(_F32)

    qr0, qc0 = _quads(vertices)
    fm0 = _call_batched(
        functools.partial(_conv0_body, n_nbr=10),
        [vertices, qr0, qc0],
        [p['conv0']['ste'].T, p['conv0']['directions']],
        (B, V, 128))

    c1 = _hs_layer(vertices, fm0, p['conv1'], 10, 128)
    fm1 = _bn_relu(c1, p['bn1_g'], p['bn1_b'])

    idx1 = np.random.RandomState(0).permutation(V)[: V // 4]
    v1 = vertices[:, idx1, :]
    f1 = _pool(v1, vertices, fm1, 4)

    n23 = min(10, (V // 4) // 8)

    c2 = _hs_layer(v1, f1, p['conv2'], n23, 256)
    fm2 = _bn_relu(c2, p['bn2_g'], p['bn2_b'])

    c3 = _hs_layer(v1, fm2, p['conv3'], n23, 256)
    fm3 = _bn_relu(c3, p['bn3_g'], p['bn3_b'])

    idx2 = np.random.RandomState(1).permutation(V // 4)[: V // 16]
    v2 = v1[:, idx2, :]
    f2 = _pool(v2, v1, fm3, 4)

    n4 = min(10, (V // 16) // 8)

    fm4 = _hs_layer(v2, f2, p['conv4'], n4, 512)

    q1 = jnp.sum(v1 * v1, axis=2)
    q2 = jnp.sum(v2 * v2, axis=2)
    qt = jnp.sum(vertices * vertices, axis=2)
    oh = jax.nn.one_hot(cat_id.reshape(-1), 6, dtype=_F32)[:, None, :]
    w1t = p['mlp_w1'].T
    h = _call_batched(
        _head_body,
        [vertices, v1, v2, q1[:, None, :], q2[:, None, :], qt[:, :, None],
         fm0, fm1, fm2, fm3, fm4, oh],
        [w1t[:128], w1t[128:256], w1t[256:512], w1t[512:768],
         w1t[768:1280], w1t[1280:1286], p['mlp_b1'].reshape(1, -1),
         p['mlp_w2'].T, p['mlp_b2'].reshape(1, -1),
         p['mlp_w3'].T, p['mlp_b3'].reshape(1, -1)],
        (B, V, 128))
    return jnp.transpose(h, (0, 2, 1))


# speedup vs baseline: 7.0746x; 1.0009x over previous
import functools

import numpy as np
import jax
import jax.numpy as jnp
from jax.experimental import pallas as pl

_SUP = 7
_EPS = 1e-12
_F32 = jnp.float32

_NT = (((1,), (1,)), ((), ()))


def _ntdot(a, b):
    return jax.lax.dot_general(a, b, _NT, preferred_element_type=_F32)


def _nndot(a, b):
    return jnp.dot(a, b, preferred_element_type=_F32)


def _gdot(onehot, b):
    oh = onehot.astype(jnp.bfloat16)
    b1 = b.astype(jnp.bfloat16)
    r1 = b - b1.astype(_F32)
    b2 = r1.astype(jnp.bfloat16)
    b3 = (r1 - b2.astype(_F32)).astype(jnp.bfloat16)
    g1 = jnp.dot(oh, b1, preferred_element_type=_F32)
    g2 = jnp.dot(oh, b2, preferred_element_type=_F32)
    g3 = jnp.dot(oh, b3, preferred_element_type=_F32)
    return (g1 + g2) + g3


def _gdot2(onehot, b):
    oh = onehot.astype(jnp.bfloat16)
    b1 = b.astype(jnp.bfloat16)
    b2 = (b - b1.astype(_F32)).astype(jnp.bfloat16)
    g1 = jnp.dot(oh, b1, preferred_element_type=_F32)
    g2 = jnp.dot(oh, b2, preferred_element_type=_F32)
    return g1 + g2


def _normalize_rows(x):
    n = jnp.sqrt(jnp.sum(x * x, axis=-1, keepdims=True))
    return x / jnp.maximum(n, _EPS)


def _pop_argmax(neg, iota):
    m = jnp.max(neg, axis=1, keepdims=True)
    eq = neg == m
    big = jnp.int32(neg.shape[1])
    amx = jnp.min(jnp.where(eq, iota, big), axis=1, keepdims=True)
    sel = iota == amx
    return sel.astype(neg.dtype), jnp.where(sel, -jnp.inf, neg)


def _neg_dist(feat_rows, feat_all, qrow, qcol):
    inner = _ntdot(feat_rows, feat_all)
    return -((-2.0 * inner + qrow) + qcol)


def _sum_supports(acc, out_ch):
    agg = acc[:, :out_ch]
    for s in range(1, _SUP):
        agg = agg + acc[:, s * out_ch:(s + 1) * out_ch]
    return agg


def _dir_normalize(dirs):
    n = jnp.sqrt(jnp.sum(dirs * dirs, axis=0, keepdims=True))
    return dirs / jnp.maximum(n, _EPS)


def _conv0_body(v_ref, qr_ref, qc_ref, steT_ref, dirs_ref, out_ref, *, n_nbr):
    v3 = v_ref[0]
    dn = _dir_normalize(dirs_ref[...])
    neg = _neg_dist(v3, v3, qr_ref[0], qc_ref[0])
    V = v3.shape[0]
    iota = jax.lax.broadcasted_iota(jnp.int32, (V, V), 1)
    _, neg = _pop_argmax(neg, iota)
    acc = jnp.full((V, dn.shape[1]), -jnp.inf, _F32)
    for _ in range(n_nbr):
        onehot, neg = _pop_argmax(neg, iota)
        nbr = _gdot(onehot, v3)
        rf = _normalize_rows(nbr - v3)
        theta = jax.nn.relu(_nndot(rf, dn))
        acc = jnp.maximum(acc, theta)
    agg = _sum_supports(acc, 128)
    f_ste = _nndot(v3, steT_ref[...])
    out_ref[0] = jax.nn.relu(agg + f_ste)


def _hs_layer_body(v_ref, fm_ref, qr_ref, qc_ref, steT_ref, dirs_ref, w_ref,
                   b_ref, out_ref, *, n_nbr, out_ch, exact=True):
    v3 = v_ref[0]
    fm = fm_ref[0]
    dn = _dir_normalize(dirs_ref[...])
    neg = _neg_dist(fm, fm, qr_ref[0], qc_ref[0])
    V = fm.shape[0]
    iota = jax.lax.broadcasted_iota(jnp.int32, (V, V), 1)
    w = w_ref[...]
    b = b_ref[...]
    f_center = _nndot(fm, w[:, :out_ch]) + b[:, :out_ch]
    w_sup = w[:, out_ch:]
    b_sup = b[:, out_ch:]
    _, neg = _pop_argmax(neg, iota)
    acc = jnp.full((V, _SUP * out_ch), -jnp.inf, _F32)
    for _ in range(n_nbr):
        onehot, neg = _pop_argmax(neg, iota)
        nbr = _gdot(onehot, v3) if exact else _gdot2(onehot, v3)
        rf = _normalize_rows(nbr - v3)
        theta = jax.nn.relu(_nndot(rf, dn))
        gfm = _gdot(onehot, fm) if exact else _gdot2(onehot, fm)
        fs = _nndot(gfm, w_sup) + b_sup
        acc = jnp.maximum(acc, theta * fs)
    agg = _sum_supports(acc, out_ch)
    f_ste = _nndot(fm, steT_ref[...])
    out_ref[0] = f_center + agg + f_ste


def _bn_relu_body(x_ref, g_ref, b_ref, out_ref):
    x = x_ref[...]
    m = jnp.mean(x, axis=0, keepdims=True)
    var = jnp.mean((x - m) ** 2, axis=0, keepdims=True)
    out_ref[...] = jax.nn.relu(
        g_ref[...] * (x - m) / jnp.sqrt(var + 1e-5) + b_ref[...])


def _pool_body(vsel_ref, v_ref, fm_ref, qr_ref, qc_ref, out_ref, *, n_nbr):
    vsel = vsel_ref[0]
    v3 = v_ref[0]
    fm = fm_ref[0]
    neg = _neg_dist(vsel, v3, qr_ref[0], qc_ref[0])
    Vo, V = neg.shape
    iota = jax.lax.broadcasted_iota(jnp.int32, (Vo, V), 1)
    _, neg = _pop_argmax(neg, iota)
    acc = jnp.full((Vo, fm.shape[1]), -jnp.inf, _F32)
    for _ in range(n_nbr):
        onehot, neg = _pop_argmax(neg, iota)
        acc = jnp.maximum(acc, _gdot(onehot, fm))
    out_ref[0] = acc


def _nearest_onehot(t, s, qr, qc):
    inner = _ntdot(t, s)
    d = (qr + qc) - 2.0 * inner
    Vt, Vs = d.shape
    iota = jax.lax.broadcasted_iota(jnp.int32, (Vt, Vs), 1)
    mn = jnp.min(d, axis=1, keepdims=True)
    amn = jnp.min(jnp.where(d == mn, iota, jnp.int32(Vs)),
                  axis=1, keepdims=True)
    return (iota == amn).astype(_F32)


def _head_body(t_ref, s1_ref, s2_ref, q1r_ref, q2r_ref, qtc_ref,
               fm0_ref, fm1_ref, fm2_ref, fm3_ref, fm4_ref, oh_ref,
               w1a_ref, w1b_ref, w1c_ref, w1d_ref, w1e_ref, w1f_ref, b1_ref,
               w2_ref, b2_ref, w3_ref, b3_ref, out_ref):
    t = t_ref[0]
    qtc = qtc_ref[0]
    oh1 = _nearest_onehot(t, s1_ref[0], q1r_ref[0], qtc)
    oh2 = _nearest_onehot(t, s2_ref[0], q2r_ref[0], qtc)
    g2 = _gdot2(oh1, fm2_ref[0])
    g3 = _gdot2(oh1, fm3_ref[0])
    g4 = _gdot2(oh2, fm4_ref[0])
    h = (_nndot(fm0_ref[0], w1a_ref[...]) + _nndot(fm1_ref[0], w1b_ref[...])
         + _nndot(g2, w1c_ref[...]) + _nndot(g3, w1d_ref[...])
         + _nndot(g4, w1e_ref[...]) + _nndot(oh_ref[0], w1f_ref[...])
         + b1_ref[...])
    h = jax.nn.relu(h)
    h = jax.nn.relu(_nndot(h, w2_ref[...]) + b2_ref[...])
    h = jax.nn.relu(_nndot(h, w3_ref[...]) + b3_ref[...])
    out_ref[0] = h


def _batched_spec(shape):
    nd = len(shape)
    return pl.BlockSpec((1,) + shape[1:], lambda b: (b,) + (0,) * (nd - 1))


def _param_spec(shape):
    nd = len(shape)
    return pl.BlockSpec(shape, lambda b: (0,) * nd)


def _call_batched(body, batched_ins, param_ins, out_shape):
    B = batched_ins[0].shape[0]
    in_specs = ([_batched_spec(x.shape) for x in batched_ins]
                + [_param_spec(p.shape) for p in param_ins])
    return pl.pallas_call(
        body,
        grid=(B,),
        in_specs=in_specs,
        out_specs=_batched_spec(out_shape),
        out_shape=jax.ShapeDtypeStruct(out_shape, _F32),
    )(*batched_ins, *param_ins)


def _quads(x):
    q = jnp.sum(x * x, axis=2)
    return q[:, None, :], q[:, :, None]


def _bn_relu(x, g, b):
    B, V, C = x.shape
    x2 = x.reshape(B * V, C)
    out = pl.pallas_call(
        _bn_relu_body,
        out_shape=jax.ShapeDtypeStruct((B * V, C), _F32),
    )(x2, g.reshape(1, C), b.reshape(1, C))
    return out.reshape(B, V, C)


def _hs_layer(v, fm, p, n_nbr, out_ch, exact=True):
    B, V, _ = v.shape
    qr, qc = _quads(fm)
    body = functools.partial(_hs_layer_body, n_nbr=n_nbr, out_ch=out_ch,
                             exact=exact)
    return _call_batched(
        body, [v, fm, qr, qc],
        [p['ste'].T, p['directions'], p['weights'],
         p['bias'].reshape(1, -1)],
        (B, V, out_ch))


def _pool(vsel, v, fm, n_nbr):
    B = v.shape[0]
    qs = jnp.sum(vsel * vsel, axis=2)
    qv = jnp.sum(v * v, axis=2)
    return _call_batched(
        functools.partial(_pool_body, n_nbr=n_nbr),
        [vsel, v, fm, qv[:, None, :], qs[:, :, None]], [],
        (B, vsel.shape[1], fm.shape[2]))


def kernel(vertices, cat_id, params):
    B, V, _ = vertices.shape
    p = params
    vertices = vertices.astype(_F32)

    qr0, qc0 = _quads(vertices)
    fm0 = _call_batched(
        functools.partial(_conv0_body, n_nbr=10),
        [vertices, qr0, qc0],
        [p['conv0']['ste'].T, p['conv0']['directions']],
        (B, V, 128))

    c1 = _hs_layer(vertices, fm0, p['conv1'], 10, 128)
    fm1 = _bn_relu(c1, p['bn1_g'], p['bn1_b'])

    idx1 = np.random.RandomState(0).permutation(V)[: V // 4]
    v1 = vertices[:, idx1, :]
    f1 = _pool(v1, vertices, fm1, 4)

    n23 = min(10, (V // 4) // 8)

    c2 = _hs_layer(v1, f1, p['conv2'], n23, 256)
    fm2 = _bn_relu(c2, p['bn2_g'], p['bn2_b'])

    c3 = _hs_layer(v1, fm2, p['conv3'], n23, 256)
    fm3 = _bn_relu(c3, p['bn3_g'], p['bn3_b'])

    idx2 = np.random.RandomState(1).permutation(V // 4)[: V // 16]
    v2 = v1[:, idx2, :]
    f2 = _pool(v2, v1, fm3, 4)

    n4 = min(10, (V // 16) // 8)

    fm4 = _hs_layer(v2, f2, p['conv4'], n4, 512, exact=False)

    q1 = jnp.sum(v1 * v1, axis=2)
    q2 = jnp.sum(v2 * v2, axis=2)
    qt = jnp.sum(vertices * vertices, axis=2)
    oh = jax.nn.one_hot(cat_id.reshape(-1), 6, dtype=_F32)[:, None, :]
    w1t = p['mlp_w1'].T
    h = _call_batched(
        _head_body,
        [vertices, v1, v2, q1[:, None, :], q2[:, None, :], qt[:, :, None],
         fm0, fm1, fm2, fm3, fm4, oh],
        [w1t[:128], w1t[128:256], w1t[256:512], w1t[512:768],
         w1t[768:1280], w1t[1280:1286], p['mlp_b1'].reshape(1, -1),
         p['mlp_w2'].T, p['mlp_b2'].reshape(1, -1),
         p['mlp_w3'].T, p['mlp_b3'].reshape(1, -1)],
        (B, V, 128))
    return jnp.transpose(h, (0, 2, 1))


# speedup vs baseline: 7.9388x; 1.1222x over previous
import functools

import numpy as np
import jax
import jax.numpy as jnp
from jax.experimental import pallas as pl

_SUP = 7
_EPS = 1e-12
_F32 = jnp.float32

_NT = (((1,), (1,)), ((), ()))


def _ntdot(a, b):
    return jax.lax.dot_general(a, b, _NT, preferred_element_type=_F32)


def _nndot(a, b):
    return jnp.dot(a, b, preferred_element_type=_F32)


def _gdot(onehot, b):
    oh = onehot.astype(jnp.bfloat16)
    b1 = b.astype(jnp.bfloat16)
    r1 = b - b1.astype(_F32)
    b2 = r1.astype(jnp.bfloat16)
    b3 = (r1 - b2.astype(_F32)).astype(jnp.bfloat16)
    g1 = jnp.dot(oh, b1, preferred_element_type=_F32)
    g2 = jnp.dot(oh, b2, preferred_element_type=_F32)
    g3 = jnp.dot(oh, b3, preferred_element_type=_F32)
    return (g1 + g2) + g3


def _gdot2(onehot, b):
    oh = onehot.astype(jnp.bfloat16)
    b1 = b.astype(jnp.bfloat16)
    b2 = (b - b1.astype(_F32)).astype(jnp.bfloat16)
    g1 = jnp.dot(oh, b1, preferred_element_type=_F32)
    g2 = jnp.dot(oh, b2, preferred_element_type=_F32)
    return g1 + g2


def _gdot_bf16(onehot, b):
    return jnp.dot(onehot.astype(jnp.bfloat16), b.astype(jnp.bfloat16),
                   preferred_element_type=_F32)


def _normalize_rows(x):
    n = jnp.sqrt(jnp.sum(x * x, axis=-1, keepdims=True))
    return x / jnp.maximum(n, _EPS)


def _pop_argmax(neg, iota):
    m = jnp.max(neg, axis=1, keepdims=True)
    eq = neg == m
    big = jnp.int32(neg.shape[1])
    amx = jnp.min(jnp.where(eq, iota, big), axis=1, keepdims=True)
    sel = iota == amx
    return sel.astype(neg.dtype), jnp.where(sel, -jnp.inf, neg)


def _neg_dist(feat_rows, feat_all, qrow, qcol):
    inner = _ntdot(feat_rows, feat_all)
    return -((-2.0 * inner + qrow) + qcol)


def _sum_supports(acc, out_ch):
    agg = acc[:, :out_ch]
    for s in range(1, _SUP):
        agg = agg + acc[:, s * out_ch:(s + 1) * out_ch]
    return agg


def _dir_normalize(dirs):
    n = jnp.sqrt(jnp.sum(dirs * dirs, axis=0, keepdims=True))
    return dirs / jnp.maximum(n, _EPS)


def _conv0_body(v_ref, qr_ref, qc_ref, steT_ref, dirs_ref, out_ref, *, n_nbr):
    v3 = v_ref[0]
    dn = _dir_normalize(dirs_ref[...])
    neg = _neg_dist(v3, v3, qr_ref[0], qc_ref[0])
    V = v3.shape[0]
    iota = jax.lax.broadcasted_iota(jnp.int32, (V, V), 1)
    _, neg = _pop_argmax(neg, iota)
    acc = jnp.full((V, dn.shape[1]), -jnp.inf, _F32)
    for _ in range(n_nbr):
        onehot, neg = _pop_argmax(neg, iota)
        nbr = _gdot(onehot, v3)
        rf = _normalize_rows(nbr - v3)
        theta = jax.nn.relu(_nndot(rf, dn))
        acc = jnp.maximum(acc, theta)
    agg = _sum_supports(acc, 128)
    f_ste = _nndot(v3, steT_ref[...])
    out_ref[0] = jax.nn.relu(agg + f_ste)


def _hs_layer_body(v_ref, fm_ref, qr_ref, qc_ref, steT_ref, dirs_ref, w_ref,
                   b_ref, out_ref, *, n_nbr, out_ch, exact=True):
    v3 = v_ref[0]
    fm = fm_ref[0]
    dn = _dir_normalize(dirs_ref[...])
    neg = _neg_dist(fm, fm, qr_ref[0], qc_ref[0])
    V = fm.shape[0]
    iota = jax.lax.broadcasted_iota(jnp.int32, (V, V), 1)
    w = w_ref[...]
    b = b_ref[...]
    f_center = _nndot(fm, w[:, :out_ch]) + b[:, :out_ch]
    w_sup = w[:, out_ch:]
    b_sup = b[:, out_ch:]
    _, neg = _pop_argmax(neg, iota)
    acc = jnp.full((V, _SUP * out_ch), -jnp.inf, _F32)
    for _ in range(n_nbr):
        onehot, neg = _pop_argmax(neg, iota)
        nbr = _gdot(onehot, v3) if exact else _gdot2(onehot, v3)
        rf = _normalize_rows(nbr - v3)
        theta = jax.nn.relu(_nndot(rf, dn))
        fs = _nndot(_gdot_bf16(onehot, fm), w_sup) + b_sup
        acc = jnp.maximum(acc, theta * fs)
    agg = _sum_supports(acc, out_ch)
    f_ste = _nndot(fm, steT_ref[...])
    out_ref[0] = f_center + agg + f_ste


def _bn_relu_body(x_ref, g_ref, b_ref, out_ref):
    x = x_ref[...]
    m = jnp.mean(x, axis=0, keepdims=True)
    var = jnp.mean((x - m) ** 2, axis=0, keepdims=True)
    out_ref[...] = jax.nn.relu(
        g_ref[...] * (x - m) / jnp.sqrt(var + 1e-5) + b_ref[...])


def _pool_body(vsel_ref, v_ref, fm_ref, qr_ref, qc_ref, out_ref, *, n_nbr):
    vsel = vsel_ref[0]
    v3 = v_ref[0]
    fm = fm_ref[0]
    neg = _neg_dist(vsel, v3, qr_ref[0], qc_ref[0])
    Vo, V = neg.shape
    iota = jax.lax.broadcasted_iota(jnp.int32, (Vo, V), 1)
    _, neg = _pop_argmax(neg, iota)
    acc = jnp.full((Vo, fm.shape[1]), -jnp.inf, _F32)
    for _ in range(n_nbr):
        onehot, neg = _pop_argmax(neg, iota)
        acc = jnp.maximum(acc, _gdot(onehot, fm))
    out_ref[0] = acc


def _nearest_onehot(t, s, qr, qc):
    inner = _ntdot(t, s)
    d = (qr + qc) - 2.0 * inner
    Vt, Vs = d.shape
    iota = jax.lax.broadcasted_iota(jnp.int32, (Vt, Vs), 1)
    mn = jnp.min(d, axis=1, keepdims=True)
    amn = jnp.min(jnp.where(d == mn, iota, jnp.int32(Vs)),
                  axis=1, keepdims=True)
    return (iota == amn).astype(_F32)


def _head_body(t_ref, s1_ref, s2_ref, q1r_ref, q2r_ref, qtc_ref,
               fm0_ref, fm1_ref, fm2_ref, fm3_ref, fm4_ref, oh_ref,
               w1a_ref, w1b_ref, w1c_ref, w1d_ref, w1e_ref, w1f_ref, b1_ref,
               w2_ref, b2_ref, w3_ref, b3_ref, out_ref):
    t = t_ref[0]
    qtc = qtc_ref[0]
    oh1 = _nearest_onehot(t, s1_ref[0], q1r_ref[0], qtc)
    oh2 = _nearest_onehot(t, s2_ref[0], q2r_ref[0], qtc)
    g2 = _gdot_bf16(oh1, fm2_ref[0])
    g3 = _gdot_bf16(oh1, fm3_ref[0])
    g4 = _gdot_bf16(oh2, fm4_ref[0])
    h = (_nndot(fm0_ref[0], w1a_ref[...]) + _nndot(fm1_ref[0], w1b_ref[...])
         + _nndot(g2, w1c_ref[...]) + _nndot(g3, w1d_ref[...])
         + _nndot(g4, w1e_ref[...]) + _nndot(oh_ref[0], w1f_ref[...])
         + b1_ref[...])
    h = jax.nn.relu(h)
    h = jax.nn.relu(_nndot(h, w2_ref[...]) + b2_ref[...])
    h = jax.nn.relu(_nndot(h, w3_ref[...]) + b3_ref[...])
    out_ref[0] = h


def _batched_spec(shape):
    nd = len(shape)
    return pl.BlockSpec((1,) + shape[1:], lambda b: (b,) + (0,) * (nd - 1))


def _param_spec(shape):
    nd = len(shape)
    return pl.BlockSpec(shape, lambda b: (0,) * nd)


def _call_batched(body, batched_ins, param_ins, out_shape):
    B = batched_ins[0].shape[0]
    in_specs = ([_batched_spec(x.shape) for x in batched_ins]
                + [_param_spec(p.shape) for p in param_ins])
    return pl.pallas_call(
        body,
        grid=(B,),
        in_specs=in_specs,
        out_specs=_batched_spec(out_shape),
        out_shape=jax.ShapeDtypeStruct(out_shape, _F32),
    )(*batched_ins, *param_ins)


def _quads(x):
    q = jnp.sum(x * x, axis=2)
    return q[:, None, :], q[:, :, None]


def _bn_relu(x, g, b):
    B, V, C = x.shape
    x2 = x.reshape(B * V, C)
    out = pl.pallas_call(
        _bn_relu_body,
        out_shape=jax.ShapeDtypeStruct((B * V, C), _F32),
    )(x2, g.reshape(1, C), b.reshape(1, C))
    return out.reshape(B, V, C)


def _hs_layer(v, fm, p, n_nbr, out_ch, exact=True):
    B, V, _ = v.shape
    qr, qc = _quads(fm)
    body = functools.partial(_hs_layer_body, n_nbr=n_nbr, out_ch=out_ch,
                             exact=exact)
    return _call_batched(
        body, [v, fm, qr, qc],
        [p['ste'].T, p['directions'], p['weights'],
         p['bias'].reshape(1, -1)],
        (B, V, out_ch))


def _pool(vsel, v, fm, n_nbr):
    B = v.shape[0]
    qs = jnp.sum(vsel * vsel, axis=2)
    qv = jnp.sum(v * v, axis=2)
    return _call_batched(
        functools.partial(_pool_body, n_nbr=n_nbr),
        [vsel, v, fm, qv[:, None, :], qs[:, :, None]], [],
        (B, vsel.shape[1], fm.shape[2]))


def kernel(vertices, cat_id, params):
    B, V, _ = vertices.shape
    p = params
    vertices = vertices.astype(_F32)

    qr0, qc0 = _quads(vertices)
    fm0 = _call_batched(
        functools.partial(_conv0_body, n_nbr=10),
        [vertices, qr0, qc0],
        [p['conv0']['ste'].T, p['conv0']['directions']],
        (B, V, 128))

    c1 = _hs_layer(vertices, fm0, p['conv1'], 10, 128)
    fm1 = _bn_relu(c1, p['bn1_g'], p['bn1_b'])

    idx1 = np.random.RandomState(0).permutation(V)[: V // 4]
    v1 = vertices[:, idx1, :]
    f1 = _pool(v1, vertices, fm1, 4)

    n23 = min(10, (V // 4) // 8)

    c2 = _hs_layer(v1, f1, p['conv2'], n23, 256)
    fm2 = _bn_relu(c2, p['bn2_g'], p['bn2_b'])

    c3 = _hs_layer(v1, fm2, p['conv3'], n23, 256)
    fm3 = _bn_relu(c3, p['bn3_g'], p['bn3_b'])

    idx2 = np.random.RandomState(1).permutation(V // 4)[: V // 16]
    v2 = v1[:, idx2, :]
    f2 = _pool(v2, v1, fm3, 4)

    n4 = min(10, (V // 16) // 8)

    fm4 = _hs_layer(v2, f2, p['conv4'], n4, 512, exact=False)

    q1 = jnp.sum(v1 * v1, axis=2)
    q2 = jnp.sum(v2 * v2, axis=2)
    qt = jnp.sum(vertices * vertices, axis=2)
    oh = jax.nn.one_hot(cat_id.reshape(-1), 6, dtype=_F32)[:, None, :]
    w1t = p['mlp_w1'].T
    h = _call_batched(
        _head_body,
        [vertices, v1, v2, q1[:, None, :], q2[:, None, :], qt[:, :, None],
         fm0, fm1, fm2, fm3, fm4, oh],
        [w1t[:128], w1t[128:256], w1t[256:512], w1t[512:768],
         w1t[768:1280], w1t[1280:1286], p['mlp_b1'].reshape(1, -1),
         p['mlp_w2'].T, p['mlp_b2'].reshape(1, -1),
         p['mlp_w3'].T, p['mlp_b3'].reshape(1, -1)],
        (B, V, 128))
    return jnp.transpose(h, (0, 2, 1))


# speedup vs baseline: 10.8711x; 1.3694x over previous
import functools

import numpy as np
import jax
import jax.numpy as jnp
from jax.experimental import pallas as pl

_SUP = 7
_EPS = 1e-12
_F32 = jnp.float32

_NT = (((1,), (1,)), ((), ()))


def _ntdot(a, b):
    return jax.lax.dot_general(a, b, _NT, preferred_element_type=_F32)


def _nndot(a, b):
    return jnp.dot(a, b, preferred_element_type=_F32)


def _gdot(onehot, b):
    oh = onehot.astype(jnp.bfloat16)
    b1 = b.astype(jnp.bfloat16)
    r1 = b - b1.astype(_F32)
    b2 = r1.astype(jnp.bfloat16)
    b3 = (r1 - b2.astype(_F32)).astype(jnp.bfloat16)
    g1 = jnp.dot(oh, b1, preferred_element_type=_F32)
    g2 = jnp.dot(oh, b2, preferred_element_type=_F32)
    g3 = jnp.dot(oh, b3, preferred_element_type=_F32)
    return (g1 + g2) + g3


def _gdot2(onehot, b):
    oh = onehot.astype(jnp.bfloat16)
    b1 = b.astype(jnp.bfloat16)
    b2 = (b - b1.astype(_F32)).astype(jnp.bfloat16)
    g1 = jnp.dot(oh, b1, preferred_element_type=_F32)
    g2 = jnp.dot(oh, b2, preferred_element_type=_F32)
    return g1 + g2


def _gdot_bf16(onehot, b):
    return jnp.dot(onehot.astype(jnp.bfloat16), b.astype(jnp.bfloat16),
                   preferred_element_type=_F32)


def _gather_coords(sel, vrows):
    cols = [jnp.sum(jnp.where(sel, vrows[i:i + 1, :], 0.0),
                    axis=1, keepdims=True) for i in range(3)]
    return jnp.concatenate(cols, axis=1)


def _normalize_rows(x):
    n = jnp.sqrt(jnp.sum(x * x, axis=-1, keepdims=True))
    return x / jnp.maximum(n, _EPS)


def _pop_argmax(neg, iota):
    m = jnp.max(neg, axis=1, keepdims=True)
    eq = neg == m
    big = jnp.int32(neg.shape[1])
    amx = jnp.min(jnp.where(eq, iota, big), axis=1, keepdims=True)
    sel = iota == amx
    return sel, jnp.where(sel, -jnp.inf, neg)


def _neg_dist(feat_rows, feat_all, qrow, qcol):
    inner = _ntdot(feat_rows, feat_all)
    return -((-2.0 * inner + qrow) + qcol)


def _sum_supports(acc, out_ch):
    agg = acc[:, :out_ch]
    for s in range(1, _SUP):
        agg = agg + acc[:, s * out_ch:(s + 1) * out_ch]
    return agg


def _dir_normalize(dirs):
    n = jnp.sqrt(jnp.sum(dirs * dirs, axis=0, keepdims=True))
    return dirs / jnp.maximum(n, _EPS)


def _conv0_body(v_ref, vT_ref, qr_ref, qc_ref, steT_ref, dirs_ref, out_ref,
                *, n_nbr):
    v3 = v_ref[0]
    vrows = vT_ref[0]
    dn = _dir_normalize(dirs_ref[...])
    neg = _neg_dist(v3, v3, qr_ref[0], qc_ref[0])
    V = v3.shape[0]
    iota = jax.lax.broadcasted_iota(jnp.int32, (V, V), 1)
    _, neg = _pop_argmax(neg, iota)
    acc = jnp.full((V, dn.shape[1]), -jnp.inf, _F32)
    for _ in range(n_nbr):
        sel, neg = _pop_argmax(neg, iota)
        nbr = _gather_coords(sel, vrows)
        rf = _normalize_rows(nbr - v3)
        theta = jax.nn.relu(_nndot(rf, dn))
        acc = jnp.maximum(acc, theta)
    agg = _sum_supports(acc, 128)
    f_ste = _nndot(v3, steT_ref[...])
    out_ref[0] = jax.nn.relu(agg + f_ste)


def _hs_layer_body(v_ref, vT_ref, fm_ref, qr_ref, qc_ref, steT_ref, dirs_ref,
                   w_ref, b_ref, out_ref, *, n_nbr, out_ch, exact=True):
    v3 = v_ref[0]
    vrows = vT_ref[0]
    fm = fm_ref[0]
    dn = _dir_normalize(dirs_ref[...])
    neg = _neg_dist(fm, fm, qr_ref[0], qc_ref[0])
    V = fm.shape[0]
    iota = jax.lax.broadcasted_iota(jnp.int32, (V, V), 1)
    w = w_ref[...]
    b = b_ref[...]
    f_center = _nndot(fm, w[:, :out_ch]) + b[:, :out_ch]
    w_sup = w[:, out_ch:]
    b_sup = b[:, out_ch:]
    _, neg = _pop_argmax(neg, iota)
    acc = jnp.full((V, _SUP * out_ch), -jnp.inf, _F32)
    for _ in range(n_nbr):
        sel, neg = _pop_argmax(neg, iota)
        nbr = _gather_coords(sel, vrows)
        rf = _normalize_rows(nbr - v3)
        theta = jax.nn.relu(_nndot(rf, dn))
        fs = _nndot(_gdot_bf16(sel.astype(_F32), fm), w_sup) + b_sup
        acc = jnp.maximum(acc, theta * fs)
    agg = _sum_supports(acc, out_ch)
    f_ste = _nndot(fm, steT_ref[...])
    out_ref[0] = f_center + agg + f_ste


def _bn_relu_body(x_ref, g_ref, b_ref, out_ref):
    x = x_ref[...]
    m = jnp.mean(x, axis=0, keepdims=True)
    var = jnp.mean((x - m) ** 2, axis=0, keepdims=True)
    out_ref[...] = jax.nn.relu(
        g_ref[...] * (x - m) / jnp.sqrt(var + 1e-5) + b_ref[...])


def _pool_body(vsel_ref, v_ref, fm_ref, qr_ref, qc_ref, out_ref, *, n_nbr):
    vsel = vsel_ref[0]
    v3 = v_ref[0]
    fm = fm_ref[0]
    neg = _neg_dist(vsel, v3, qr_ref[0], qc_ref[0])
    Vo, V = neg.shape
    iota = jax.lax.broadcasted_iota(jnp.int32, (Vo, V), 1)
    _, neg = _pop_argmax(neg, iota)
    acc = jnp.full((Vo, fm.shape[1]), -jnp.inf, _F32)
    for _ in range(n_nbr):
        sel, neg = _pop_argmax(neg, iota)
        acc = jnp.maximum(acc, _gdot(sel.astype(_F32), fm))
    out_ref[0] = acc


def _nearest_onehot(t, s, qr, qc):
    inner = _ntdot(t, s)
    d = (qr + qc) - 2.0 * inner
    Vt, Vs = d.shape
    iota = jax.lax.broadcasted_iota(jnp.int32, (Vt, Vs), 1)
    mn = jnp.min(d, axis=1, keepdims=True)
    amn = jnp.min(jnp.where(d == mn, iota, jnp.int32(Vs)),
                  axis=1, keepdims=True)
    return (iota == amn).astype(_F32)


def _head_body(t_ref, s1_ref, s2_ref, q1r_ref, q2r_ref, qtc_ref,
               fm0_ref, fm1_ref, fm2_ref, fm3_ref, fm4_ref, oh_ref,
               w1a_ref, w1b_ref, w1c_ref, w1d_ref, w1e_ref, w1f_ref, b1_ref,
               w2_ref, b2_ref, w3_ref, b3_ref, out_ref):
    t = t_ref[0]
    qtc = qtc_ref[0]
    oh1 = _nearest_onehot(t, s1_ref[0], q1r_ref[0], qtc)
    oh2 = _nearest_onehot(t, s2_ref[0], q2r_ref[0], qtc)
    g2 = _gdot_bf16(oh1, fm2_ref[0])
    g3 = _gdot_bf16(oh1, fm3_ref[0])
    g4 = _gdot_bf16(oh2, fm4_ref[0])
    h = (_nndot(fm0_ref[0], w1a_ref[...]) + _nndot(fm1_ref[0], w1b_ref[...])
         + _nndot(g2, w1c_ref[...]) + _nndot(g3, w1d_ref[...])
         + _nndot(g4, w1e_ref[...]) + _nndot(oh_ref[0], w1f_ref[...])
         + b1_ref[...])
    h = jax.nn.relu(h)
    h = jax.nn.relu(_nndot(h, w2_ref[...]) + b2_ref[...])
    h = jax.nn.relu(_nndot(h, w3_ref[...]) + b3_ref[...])
    out_ref[0] = h


def _batched_spec(shape):
    nd = len(shape)
    return pl.BlockSpec((1,) + shape[1:], lambda b: (b,) + (0,) * (nd - 1))


def _param_spec(shape):
    nd = len(shape)
    return pl.BlockSpec(shape, lambda b: (0,) * nd)


def _call_batched(body, batched_ins, param_ins, out_shape):
    B = batched_ins[0].shape[0]
    in_specs = ([_batched_spec(x.shape) for x in batched_ins]
                + [_param_spec(p.shape) for p in param_ins])
    return pl.pallas_call(
        body,
        grid=(B,),
        in_specs=in_specs,
        out_specs=_batched_spec(out_shape),
        out_shape=jax.ShapeDtypeStruct(out_shape, _F32),
    )(*batched_ins, *param_ins)


def _quads(x):
    q = jnp.sum(x * x, axis=2)
    return q[:, None, :], q[:, :, None]


def _bn_relu(x, g, b):
    B, V, C = x.shape
    x2 = x.reshape(B * V, C)
    out = pl.pallas_call(
        _bn_relu_body,
        out_shape=jax.ShapeDtypeStruct((B * V, C), _F32),
    )(x2, g.reshape(1, C), b.reshape(1, C))
    return out.reshape(B, V, C)


def _vt8(v):
    vt = jnp.transpose(v, (0, 2, 1))
    B, _, V = vt.shape
    return jnp.concatenate([vt, jnp.zeros((B, 5, V), _F32)], axis=1)


def _hs_layer(v, fm, p, n_nbr, out_ch, exact=True):
    B, V, _ = v.shape
    qr, qc = _quads(fm)
    body = functools.partial(_hs_layer_body, n_nbr=n_nbr, out_ch=out_ch,
                             exact=exact)
    return _call_batched(
        body, [v, _vt8(v), fm, qr, qc],
        [p['ste'].T, p['directions'], p['weights'],
         p['bias'].reshape(1, -1)],
        (B, V, out_ch))


def _pool(vsel, v, fm, n_nbr):
    B = v.shape[0]
    qs = jnp.sum(vsel * vsel, axis=2)
    qv = jnp.sum(v * v, axis=2)
    return _call_batched(
        functools.partial(_pool_body, n_nbr=n_nbr),
        [vsel, v, fm, qv[:, None, :], qs[:, :, None]], [],
        (B, vsel.shape[1], fm.shape[2]))


def kernel(vertices, cat_id, params):
    B, V, _ = vertices.shape
    p = params
    vertices = vertices.astype(_F32)

    qr0, qc0 = _quads(vertices)
    fm0 = _call_batched(
        functools.partial(_conv0_body, n_nbr=10),
        [vertices, _vt8(vertices), qr0, qc0],
        [p['conv0']['ste'].T, p['conv0']['directions']],
        (B, V, 128))

    c1 = _hs_layer(vertices, fm0, p['conv1'], 10, 128)
    fm1 = _bn_relu(c1, p['bn1_g'], p['bn1_b'])

    idx1 = np.random.RandomState(0).permutation(V)[: V // 4]
    v1 = vertices[:, idx1, :]
    f1 = _pool(v1, vertices, fm1, 4)

    n23 = min(10, (V // 4) // 8)

    c2 = _hs_layer(v1, f1, p['conv2'], n23, 256)
    fm2 = _bn_relu(c2, p['bn2_g'], p['bn2_b'])

    c3 = _hs_layer(v1, fm2, p['conv3'], n23, 256)
    fm3 = _bn_relu(c3, p['bn3_g'], p['bn3_b'])

    idx2 = np.random.RandomState(1).permutation(V // 4)[: V // 16]
    v2 = v1[:, idx2, :]
    f2 = _pool(v2, v1, fm3, 4)

    n4 = min(10, (V // 16) // 8)

    fm4 = _hs_layer(v2, f2, p['conv4'], n4, 512, exact=False)

    q1 = jnp.sum(v1 * v1, axis=2)
    q2 = jnp.sum(v2 * v2, axis=2)
    qt = jnp.sum(vertices * vertices, axis=2)
    oh = jax.nn.one_hot(cat_id.reshape(-1), 6, dtype=_F32)[:, None, :]
    w1t = p['mlp_w1'].T
    h = _call_batched(
        _head_body,
        [vertices, v1, v2, q1[:, None, :], q2[:, None, :], qt[:, :, None],
         fm0, fm1, fm2, fm3, fm4, oh],
        [w1t[:128], w1t[128:256], w1t[256:512], w1t[512:768],
         w1t[768:1280], w1t[1280:1286], p['mlp_b1'].reshape(1, -1),
         p['mlp_w2'].T, p['mlp_b2'].reshape(1, -1),
         p['mlp_w3'].T, p['mlp_b3'].reshape(1, -1)],
        (B, V, 128))
    return jnp.transpose(h, (0, 2, 1))


# speedup vs baseline: 10.8761x; 1.0005x over previous
import functools

import numpy as np
import jax
import jax.numpy as jnp
from jax.experimental import pallas as pl

_SUP = 7
_EPS = 1e-12
_F32 = jnp.float32

_NT = (((1,), (1,)), ((), ()))


def _ntdot(a, b):
    return jax.lax.dot_general(a, b, _NT, preferred_element_type=_F32)


def _nndot(a, b):
    return jnp.dot(a, b, preferred_element_type=_F32)


def _gdot(onehot, b):
    oh = onehot.astype(jnp.bfloat16)
    b1 = b.astype(jnp.bfloat16)
    r1 = b - b1.astype(_F32)
    b2 = r1.astype(jnp.bfloat16)
    b3 = (r1 - b2.astype(_F32)).astype(jnp.bfloat16)
    g1 = jnp.dot(oh, b1, preferred_element_type=_F32)
    g2 = jnp.dot(oh, b2, preferred_element_type=_F32)
    g3 = jnp.dot(oh, b3, preferred_element_type=_F32)
    return (g1 + g2) + g3


def _gdot_bf16(onehot, b):
    return jnp.dot(onehot.astype(jnp.bfloat16), b.astype(jnp.bfloat16),
                   preferred_element_type=_F32)


def _gather_coords(sel, vrows):
    cols = [jnp.sum(jnp.where(sel, vrows[i:i + 1, :], 0.0),
                    axis=1, keepdims=True) for i in range(3)]
    return jnp.concatenate(cols, axis=1)


def _normalize_rows(x):
    n = jnp.sqrt(jnp.sum(x * x, axis=-1, keepdims=True))
    return x / jnp.maximum(n, _EPS)


def _pop_argmax(neg, iota):
    m = jnp.max(neg, axis=1, keepdims=True)
    eq = neg == m
    big = jnp.int32(neg.shape[1])
    amx = jnp.min(jnp.where(eq, iota, big), axis=1, keepdims=True)
    sel = iota == amx
    return sel, jnp.where(sel, -jnp.inf, neg)


def _neg_dist(feat_rows, feat_all, qrow, qcol):
    inner = _ntdot(feat_rows, feat_all)
    return -((-2.0 * inner + qrow) + qcol)


def _sum_supports(acc, out_ch):
    agg = acc[:, :out_ch]
    for s in range(1, _SUP):
        agg = agg + acc[:, s * out_ch:(s + 1) * out_ch]
    return agg


def _dir_normalize(dirs):
    n = jnp.sqrt(jnp.sum(dirs * dirs, axis=0, keepdims=True))
    return dirs / jnp.maximum(n, _EPS)


def _conv0_body(v_ref, vT_ref, qr_ref, qc_ref, steT_ref, dirs_ref, out_ref,
                *, n_nbr):
    v3 = v_ref[0]
    vrows = vT_ref[0]
    dn = _dir_normalize(dirs_ref[...])
    neg = _neg_dist(v3, v3, qr_ref[0], qc_ref[0])
    V = v3.shape[0]
    iota = jax.lax.broadcasted_iota(jnp.int32, (V, V), 1)
    _, neg = _pop_argmax(neg, iota)
    acc = jnp.full((V, dn.shape[1]), -jnp.inf, _F32)
    for _ in range(n_nbr):
        sel, neg = _pop_argmax(neg, iota)
        nbr = _gather_coords(sel, vrows)
        rf = _normalize_rows(nbr - v3)
        theta = jax.nn.relu(_nndot(rf, dn))
        acc = jnp.maximum(acc, theta)
    agg = _sum_supports(acc, 128)
    f_ste = _nndot(v3, steT_ref[...])
    out_ref[0] = jax.nn.relu(agg + f_ste)


def _hs_layer_body(v_ref, vT_ref, fm_ref, qr_ref, qc_ref, steT_ref, dirs_ref,
                   w_ref, b_ref, out_ref, *, n_nbr, out_ch):
    v3 = v_ref[0]
    vrows = vT_ref[0]
    fm = fm_ref[0]
    dn = _dir_normalize(dirs_ref[...])
    neg = _neg_dist(fm, fm, qr_ref[0], qc_ref[0])
    V = fm.shape[0]
    iota = jax.lax.broadcasted_iota(jnp.int32, (V, V), 1)
    w = w_ref[...]
    b = b_ref[...]
    f_center = _nndot(fm, w[:, :out_ch]) + b[:, :out_ch]
    w_sup = w[:, out_ch:]
    b_sup = b[:, out_ch:]
    _, neg = _pop_argmax(neg, iota)
    acc = jnp.full((V, _SUP * out_ch), -jnp.inf, _F32)
    for _ in range(n_nbr):
        sel, neg = _pop_argmax(neg, iota)
        nbr = _gather_coords(sel, vrows)
        rf = _normalize_rows(nbr - v3)
        theta = jax.nn.relu(_nndot(rf, dn))
        fs = _nndot(_gdot_bf16(sel.astype(_F32), fm), w_sup) + b_sup
        acc = jnp.maximum(acc, theta * fs)
    agg = _sum_supports(acc, out_ch)
    f_ste = _nndot(fm, steT_ref[...])
    out_ref[0] = f_center + agg + f_ste


def _bn_relu_body(x_ref, g_ref, b_ref, out_ref):
    x = x_ref[...]
    m = jnp.mean(x, axis=0, keepdims=True)
    var = jnp.mean((x - m) ** 2, axis=0, keepdims=True)
    out_ref[...] = jax.nn.relu(
        g_ref[...] * (x - m) / jnp.sqrt(var + 1e-5) + b_ref[...])


def _pool_body(vsel_ref, v_ref, fm_ref, qr_ref, qc_ref, out_ref, *, n_nbr):
    vsel = vsel_ref[0]
    v3 = v_ref[0]
    fm = fm_ref[0]
    neg = _neg_dist(vsel, v3, qr_ref[0], qc_ref[0])
    Vo, V = neg.shape
    iota = jax.lax.broadcasted_iota(jnp.int32, (Vo, V), 1)
    _, neg = _pop_argmax(neg, iota)
    acc = jnp.full((Vo, fm.shape[1]), -jnp.inf, _F32)
    for _ in range(n_nbr):
        sel, neg = _pop_argmax(neg, iota)
        acc = jnp.maximum(acc, _gdot(sel.astype(_F32), fm))
    out_ref[0] = acc


def _nearest_onehot(t, s, qr, qc):
    inner = _ntdot(t, s)
    d = (qr + qc) - 2.0 * inner
    Vt, Vs = d.shape
    iota = jax.lax.broadcasted_iota(jnp.int32, (Vt, Vs), 1)
    mn = jnp.min(d, axis=1, keepdims=True)
    amn = jnp.min(jnp.where(d == mn, iota, jnp.int32(Vs)),
                  axis=1, keepdims=True)
    return (iota == amn).astype(_F32)


def _head_body(t_ref, s1_ref, s2_ref, q1r_ref, q2r_ref, qtc_ref,
               fm0_ref, fm1_ref, fm2_ref, fm3_ref, fm4_ref, oh_ref,
               w1a_ref, w1b_ref, w1c_ref, w1d_ref, w1e_ref, w1f_ref, b1_ref,
               w2_ref, b2_ref, w3_ref, b3_ref, out_ref):
    t = t_ref[0]
    qtc = qtc_ref[0]
    oh1 = _nearest_onehot(t, s1_ref[0], q1r_ref[0], qtc)
    oh2 = _nearest_onehot(t, s2_ref[0], q2r_ref[0], qtc)
    g2 = _gdot_bf16(oh1, fm2_ref[0])
    g3 = _gdot_bf16(oh1, fm3_ref[0])
    g4 = _gdot_bf16(oh2, fm4_ref[0])
    h = (_nndot(fm0_ref[0], w1a_ref[...]) + _nndot(fm1_ref[0], w1b_ref[...])
         + _nndot(g2, w1c_ref[...]) + _nndot(g3, w1d_ref[...])
         + _nndot(g4, w1e_ref[...]) + _nndot(oh_ref[0], w1f_ref[...])
         + b1_ref[...])
    h = jax.nn.relu(h)
    h = jax.nn.relu(_nndot(h, w2_ref[...]) + b2_ref[...])
    h = jax.nn.relu(_nndot(h, w3_ref[...]) + b3_ref[...])
    out_ref[0] = h


def _batched_spec(shape):
    nd = len(shape)
    return pl.BlockSpec((1,) + shape[1:], lambda b: (b,) + (0,) * (nd - 1))


def _param_spec(shape):
    nd = len(shape)
    return pl.BlockSpec(shape, lambda b: (0,) * nd)


def _call_batched(body, batched_ins, param_ins, out_shape):
    B = batched_ins[0].shape[0]
    in_specs = ([_batched_spec(x.shape) for x in batched_ins]
                + [_param_spec(p.shape) for p in param_ins])
    return pl.pallas_call(
        body,
        grid=(B,),
        in_specs=in_specs,
        out_specs=_batched_spec(out_shape),
        out_shape=jax.ShapeDtypeStruct(out_shape, _F32),
    )(*batched_ins, *param_ins)


def _quads(x):
    q = jnp.sum(x * x, axis=2)
    return q[:, None, :], q[:, :, None]


def _bn_relu(x, g, b):
    B, V, C = x.shape
    x2 = x.reshape(B * V, C)
    out = pl.pallas_call(
        _bn_relu_body,
        out_shape=jax.ShapeDtypeStruct((B * V, C), _F32),
    )(x2, g.reshape(1, C), b.reshape(1, C))
    return out.reshape(B, V, C)


def _vt8(v):
    vt = jnp.transpose(v, (0, 2, 1))
    B, _, V = vt.shape
    return jnp.concatenate([vt, jnp.zeros((B, 5, V), _F32)], axis=1)


def _hs_layer(v, fm, p, n_nbr, out_ch):
    B, V, _ = v.shape
    qr, qc = _quads(fm)
    body = functools.partial(_hs_layer_body, n_nbr=n_nbr, out_ch=out_ch)
    return _call_batched(
        body, [v, _vt8(v), fm, qr, qc],
        [p['ste'].T, p['directions'], p['weights'],
         p['bias'].reshape(1, -1)],
        (B, V, out_ch))


def _pool(vsel, v, fm, n_nbr):
    B = v.shape[0]
    qs = jnp.sum(vsel * vsel, axis=2)
    qv = jnp.sum(v * v, axis=2)
    return _call_batched(
        functools.partial(_pool_body, n_nbr=n_nbr),
        [vsel, v, fm, qv[:, None, :], qs[:, :, None]], [],
        (B, vsel.shape[1], fm.shape[2]))


def kernel(vertices, cat_id, params):
    B, V, _ = vertices.shape
    p = params
    vertices = vertices.astype(_F32)

    qr0, qc0 = _quads(vertices)
    fm0 = _call_batched(
        functools.partial(_conv0_body, n_nbr=10),
        [vertices, _vt8(vertices), qr0, qc0],
        [p['conv0']['ste'].T, p['conv0']['directions']],
        (B, V, 128))

    c1 = _hs_layer(vertices, fm0, p['conv1'], 10, 128)
    fm1 = _bn_relu(c1, p['bn1_g'], p['bn1_b'])

    idx1 = np.random.RandomState(0).permutation(V)[: V // 4]
    v1 = vertices[:, idx1, :]
    f1 = _pool(v1, vertices, fm1, 4)

    n23 = min(10, (V // 4) // 8)

    c2 = _hs_layer(v1, f1, p['conv2'], n23, 256)
    fm2 = _bn_relu(c2, p['bn2_g'], p['bn2_b'])

    c3 = _hs_layer(v1, fm2, p['conv3'], n23, 256)
    fm3 = _bn_relu(c3, p['bn3_g'], p['bn3_b'])

    idx2 = np.random.RandomState(1).permutation(V // 4)[: V // 16]
    v2 = v1[:, idx2, :]
    f2 = _pool(v2, v1, fm3, 4)

    n4 = min(10, (V // 16) // 8)

    fm4 = _hs_layer(v2, f2, p['conv4'], n4, 512)

    q1 = jnp.sum(v1 * v1, axis=2)
    q2 = jnp.sum(v2 * v2, axis=2)
    qt = jnp.sum(vertices * vertices, axis=2)
    oh = jax.nn.one_hot(cat_id.reshape(-1), 6, dtype=_F32)[:, None, :]
    w1t = p['mlp_w1'].T
    h = _call_batched(
        _head_body,
        [vertices, v1, v2, q1[:, None, :], q2[:, None, :], qt[:, :, None],
         fm0, fm1, fm2, fm3, fm4, oh],
        [w1t[:128], w1t[128:256], w1t[256:512], w1t[512:768],
         w1t[768:1280], w1t[1280:1286], p['mlp_b1'].reshape(1, -1),
         p['mlp_w2'].T, p['mlp_b2'].reshape(1, -1),
         p['mlp_w3'].T, p['mlp_b3'].reshape(1, -1)],
        (B, V, 128))
    return jnp.transpose(h, (0, 2, 1))
